# baseline (speedup 1.0000x reference)
import jax
import jax.numpy as jnp
from jax.experimental import pallas as pl
from jax.experimental.pallas import tpu as pltpu

_BM = 8192
_M = 100000
_STEPS = 13
_REM = _M - (_STEPS - 1) * _BM


def _mlp_block(x_ref, w1_ref, b1_ref, w2t_ref, b2_ref, o_ref):
    i = pl.program_id(0)
    h = jnp.dot(x_ref[...], w1_ref[...], preferred_element_type=jnp.float32)
    h = jnp.maximum(h + b1_ref[...], 0.0)
    ot = jax.lax.dot_general(
        w2t_ref[...], h, (((1,), (1,)), ((), ())),
        preferred_element_type=jnp.float32,
    )
    ot = ot + b2_ref[...]

    @pl.when(i < _STEPS - 1)
    def _():
        o_ref[:, pl.ds(i * _BM, _BM)] = ot

    @pl.when(i == _STEPS - 1)
    def _():
        o_ref[:, pl.ds((_STEPS - 1) * _BM, _REM)] = ot[:, :_REM]


def kernel(features, W1, b1, W2, b2):
    m, d = features.shape
    d_hid = W1.shape[1]
    n_cls = W2.shape[1]
    out_t = pl.pallas_call(
        _mlp_block,
        grid=(_STEPS,),
        in_specs=[
            pl.BlockSpec((_BM, d), lambda i: (i, 0)),
            pl.BlockSpec((d, d_hid), lambda i: (0, 0)),
            pl.BlockSpec((1, d_hid), lambda i: (0, 0)),
            pl.BlockSpec((n_cls, d_hid), lambda i: (0, 0)),
            pl.BlockSpec((n_cls, 1), lambda i: (0, 0)),
        ],
        out_specs=pl.BlockSpec((n_cls, m), lambda i: (0, 0)),
        out_shape=jax.ShapeDtypeStruct((n_cls, m), jnp.float32),
        compiler_params=pltpu.CompilerParams(
            dimension_semantics=("arbitrary",),
        ),
    )(features, W1, b1.reshape(1, -1), W2.T, b2.reshape(-1, 1))
    return out_t.T


# speedup vs baseline: 1.0286x; 1.0286x over previous
import jax
import jax.numpy as jnp
from jax.experimental import pallas as pl
from jax.experimental.pallas import tpu as pltpu

_BM = 8192
_M = 100000
_STEPS = 13
_REM = _M - (_STEPS - 1) * _BM


def _mlp_block(x_ref, w1_ref, b1_ref, w2t_ref, b2_ref, o_ref, obuf, orem, osem):
    i = pl.program_id(0)
    slot = jax.lax.rem(i, 2)

    @pl.when(i >= 2)
    def _():
        pltpu.make_async_copy(
            obuf.at[slot],
            o_ref.at[:, pl.ds((i - 2) * _BM, _BM)],
            osem.at[slot],
        ).wait()

    h = jnp.dot(x_ref[...], w1_ref[...], preferred_element_type=jnp.float32)
    h = jnp.maximum(h + b1_ref[...], 0.0)
    ot = jax.lax.dot_general(
        w2t_ref[...], h, (((1,), (1,)), ((), ())),
        preferred_element_type=jnp.float32,
    )
    @pl.when(i < _STEPS - 1)
    def _():
        obuf[slot] = ot + b2_ref[...]
        pltpu.make_async_copy(
            obuf.at[slot],
            o_ref.at[:, pl.ds(i * _BM, _BM)],
            osem.at[slot],
        ).start()

    @pl.when(i == _STEPS - 1)
    def _():
        orem[...] = (ot + b2_ref[...])[:, :_REM]
        pltpu.make_async_copy(
            orem,
            o_ref.at[:, pl.ds((_STEPS - 1) * _BM, _REM)],
            osem.at[slot],
        ).start()
        pltpu.make_async_copy(
            obuf.at[1 - slot],
            o_ref.at[:, pl.ds((i - 1) * _BM, _BM)],
            osem.at[1 - slot],
        ).wait()
        pltpu.make_async_copy(
            orem,
            o_ref.at[:, pl.ds((_STEPS - 1) * _BM, _REM)],
            osem.at[slot],
        ).wait()


def kernel(features, W1, b1, W2, b2):
    m, d = features.shape
    d_hid = W1.shape[1]
    n_cls = W2.shape[1]
    out_t = pl.pallas_call(
        _mlp_block,
        grid=(_STEPS,),
        in_specs=[
            pl.BlockSpec((_BM, d), lambda i: (i, 0)),
            pl.BlockSpec((d, d_hid), lambda i: (0, 0)),
            pl.BlockSpec((1, d_hid), lambda i: (0, 0)),
            pl.BlockSpec((n_cls, d_hid), lambda i: (0, 0)),
            pl.BlockSpec((n_cls, 1), lambda i: (0, 0)),
        ],
        out_specs=pl.BlockSpec(memory_space=pltpu.MemorySpace.HBM),
        out_shape=jax.ShapeDtypeStruct((n_cls, m), jnp.float32),
        scratch_shapes=[
            pltpu.VMEM((2, n_cls, _BM), jnp.float32),
            pltpu.VMEM((n_cls, _REM), jnp.float32),
            pltpu.SemaphoreType.DMA((2,)),
        ],
        compiler_params=pltpu.CompilerParams(
            dimension_semantics=("arbitrary",),
        ),
    )(features, W1, b1.reshape(1, -1), W2.T, b2.reshape(-1, 1))
    return out_t.T
